# baseline (speedup 1.0000x reference)
import jax
import jax.numpy as jnp
from jax import lax
from jax.experimental import pallas as pl
from jax.experimental.pallas import tpu as pltpu

EPS = 1e-5
WCHUNKS = 4


def _w_copy(hbm, vmem, sems, d, slot, midx):
    rows = hbm.shape[1]
    c = rows // WCHUNKS
    return [
        pltpu.make_async_copy(
            hbm.at[d, pl.ds(k * c, c)],
            vmem.at[slot, pl.ds(k * c, c)],
            sems.at[slot, midx, k])
        for k in range(WCHUNKS)
    ]


def _body(perm_ref, ustep_ref, first_ref, uday_ref, nuniq_ref,
          x_hbm, W1_hbm, b1_ref, W2_hbm, b2_ref, g_ref, be_ref, out_hbm,
          Xs, Ys, W1s, W2s, W1b, W2b, xsem, ysem, wsem):
    B = x_hbm.shape[0]
    nu = nuniq_ref[0]

    pltpu.make_async_copy(x_hbm.at[perm_ref[0]], Xs.at[0], xsem.at[0]).start()
    pltpu.make_async_copy(x_hbm.at[perm_ref[1]], Xs.at[1], xsem.at[1]).start()
    d0 = uday_ref[0]
    for cp in _w_copy(W1_hbm, W1s, wsem, d0, 0, 0):
        cp.start()
    for cp in _w_copy(W2_hbm, W2s, wsem, d0, 0, 1):
        cp.start()

    def step(s, carry):
        p = ustep_ref[s]
        slot = lax.rem(p, 2)
        xslot = lax.rem(s, 3)

        @pl.when(s + 2 < B)
        def _():
            pltpu.make_async_copy(x_hbm.at[perm_ref[s + 2]],
                                  Xs.at[lax.rem(s + 2, 3)],
                                  xsem.at[lax.rem(s + 2, 3)]).start()

        is_first = first_ref[s] == 1

        @pl.when(is_first)
        def _():
            d = uday_ref[p]
            for cp in _w_copy(W1_hbm, W1s, wsem, d, slot, 0):
                cp.wait()
            for cp in _w_copy(W2_hbm, W2s, wsem, d, slot, 1):
                cp.wait()
            W1b[slot] = W1s[slot].astype(jnp.bfloat16)
            W2b[slot] = W2s[slot].astype(jnp.bfloat16)

        @pl.when(is_first & (p + 1 < nu))
        def _():
            dn = uday_ref[p + 1]
            nslot = 1 - slot
            for cp in _w_copy(W1_hbm, W1s, wsem, dn, nslot, 0):
                cp.start()
            for cp in _w_copy(W2_hbm, W2s, wsem, dn, nslot, 1):
                cp.start()

        pltpu.make_async_copy(x_hbm.at[perm_ref[s]], Xs.at[xslot],
                              xsem.at[xslot]).wait()

        @pl.when(s >= 3)
        def _():
            pltpu.make_async_copy(Ys.at[xslot], out_hbm.at[perm_ref[s - 3]],
                                  ysem.at[xslot]).wait()

        d = uday_ref[p]
        xb = Xs[xslot].astype(jnp.bfloat16)
        h = jnp.dot(xb, W1b[slot],
                    preferred_element_type=jnp.float32)
        h = jnp.maximum(h + b1_ref[d], 0.0).astype(jnp.bfloat16)
        y = jnp.dot(h, W2b[slot],
                    preferred_element_type=jnp.float32)
        y = y + b2_ref[d]
        mu = jnp.mean(y, axis=-1, keepdims=True)
        yc = y - mu
        var = jnp.mean(yc * yc, axis=-1, keepdims=True)
        Ys[xslot] = yc * lax.rsqrt(var + EPS) * g_ref[d] + be_ref[d]

        pltpu.make_async_copy(Ys.at[xslot], out_hbm.at[perm_ref[s]],
                              ysem.at[xslot]).start()
        return carry

    lax.fori_loop(0, B, step, 0, unroll=False)

    for k in range(3):
        s = B - 3 + k
        pltpu.make_async_copy(Ys.at[lax.rem(s, 3)],
                              out_hbm.at[perm_ref[s]],
                              ysem.at[lax.rem(s, 3)]).wait()


def kernel(x, day_indicies, W1, b1, W2, b2, gamma, beta):
    B, T, IN = x.shape
    D, _, HID = W1.shape
    OUT = W2.shape[2]

    day = day_indicies.astype(jnp.int32)
    perm = jnp.argsort(day).astype(jnp.int32)
    sdays = jnp.take(day, perm)

    first = jnp.concatenate(
        [jnp.ones((1,), jnp.int32),
         (sdays[1:] != sdays[:-1]).astype(jnp.int32)])
    ustep = jnp.cumsum(first) - 1
    uday = jnp.zeros((B,), jnp.int32).at[ustep].set(sdays)
    nuniq = jnp.sum(first).reshape(1)

    b1r = b1.reshape(D, 1, HID)
    b2r = b2.reshape(D, 1, OUT)
    gr = gamma.reshape(D, 1, OUT)
    br = beta.reshape(D, 1, OUT)

    vec_spec = pl.BlockSpec(memory_space=pltpu.MemorySpace.VMEM)
    hbm = pl.BlockSpec(memory_space=pltpu.MemorySpace.HBM)

    grid_spec = pltpu.PrefetchScalarGridSpec(
        num_scalar_prefetch=5,
        grid=(1,),
        in_specs=[hbm, hbm, vec_spec, hbm, vec_spec, vec_spec, vec_spec],
        out_specs=hbm,
        scratch_shapes=[
            pltpu.VMEM((3, T, IN), jnp.float32),
            pltpu.VMEM((3, T, OUT), jnp.float32),
            pltpu.VMEM((2, IN, HID), jnp.float32),
            pltpu.VMEM((2, HID, OUT), jnp.float32),
            pltpu.VMEM((2, IN, HID), jnp.bfloat16),
            pltpu.VMEM((2, HID, OUT), jnp.bfloat16),
            pltpu.SemaphoreType.DMA((3,)),
            pltpu.SemaphoreType.DMA((3,)),
            pltpu.SemaphoreType.DMA((2, 2, WCHUNKS)),
        ],
    )

    return pl.pallas_call(
        _body,
        grid_spec=grid_spec,
        out_shape=jax.ShapeDtypeStruct((B, T, OUT), jnp.float32),
        compiler_params=pltpu.CompilerParams(
            dimension_semantics=("arbitrary",),
        ),
    )(perm, ustep, first, uday, nuniq,
      x, W1, b1r, W2, b2r, gr, br)


# speedup vs baseline: 1.0328x; 1.0328x over previous
import jax
import jax.numpy as jnp
from jax import lax
from jax.experimental import pallas as pl
from jax.experimental.pallas import tpu as pltpu

EPS = 1e-5
WCHUNKS = 4


def _w_copy(hbm, vmem, sems, d, slot, midx):
    rows = hbm.shape[1]
    c = rows // WCHUNKS
    return [
        pltpu.make_async_copy(
            hbm.at[d, pl.ds(k * c, c)],
            vmem.at[slot, pl.ds(k * c, c)],
            sems.at[slot, midx, k])
        for k in range(WCHUNKS)
    ]


def _body(perm_ref, ustep_ref, first_ref, uday_ref, nuniq_ref,
          x_hbm, W1_hbm, b1_ref, W2_hbm, b2_ref, g_ref, be_ref, out_hbm,
          Xs, Ys, W1s, W2s, xsem, ysem, wsem):
    B = x_hbm.shape[0]
    NP = B // 2
    nu = nuniq_ref[0]

    for k in range(2):
        for h in range(2):
            pltpu.make_async_copy(x_hbm.at[perm_ref[2 * k + h]],
                                  Xs.at[k, h], xsem.at[k, h]).start()
    d0 = uday_ref[0]
    for cp in _w_copy(W1_hbm, W1s, wsem, d0, 0, 0):
        cp.start()
    for cp in _w_copy(W2_hbm, W2s, wsem, d0, 0, 1):
        cp.start()

    def weight_logic(s):
        p = ustep_ref[s]
        slot = lax.rem(p, 2)
        is_first = first_ref[s] == 1

        @pl.when(is_first)
        def _():
            d = uday_ref[p]
            for cp in _w_copy(W1_hbm, W1s, wsem, d, slot, 0):
                cp.wait()
            for cp in _w_copy(W2_hbm, W2s, wsem, d, slot, 1):
                cp.wait()

        @pl.when(is_first & (p + 1 < nu))
        def _():
            dn = uday_ref[p + 1]
            nslot = 1 - slot
            for cp in _w_copy(W1_hbm, W1s, wsem, dn, nslot, 0):
                cp.start()
            for cp in _w_copy(W2_hbm, W2s, wsem, dn, nslot, 1):
                cp.start()
        return p, slot

    def compute(s, xslot, h, slot, p):
        d = uday_ref[p]
        xb = Xs[xslot, h].astype(jnp.bfloat16)
        hh = jnp.dot(xb, W1s[slot].astype(jnp.bfloat16),
                     preferred_element_type=jnp.float32)
        hh = jnp.maximum(hh + b1_ref[d], 0.0).astype(jnp.bfloat16)
        y = jnp.dot(hh, W2s[slot].astype(jnp.bfloat16),
                    preferred_element_type=jnp.float32)
        y = y + b2_ref[d]
        mu = jnp.mean(y, axis=-1, keepdims=True)
        yc = y - mu
        var = jnp.mean(yc * yc, axis=-1, keepdims=True)
        Ys[xslot, h] = yc * lax.rsqrt(var + EPS) * g_ref[d] + be_ref[d]
        pltpu.make_async_copy(Ys.at[xslot, h], out_hbm.at[perm_ref[s]],
                              ysem.at[xslot, h]).start()

    def step(k, carry):
        xslot = lax.rem(k, 3)

        @pl.when(k + 2 < NP)
        def _():
            ns = lax.rem(k + 2, 3)
            for h in range(2):
                pltpu.make_async_copy(x_hbm.at[perm_ref[2 * (k + 2) + h]],
                                      Xs.at[ns, h], xsem.at[ns, h]).start()

        @pl.when(k >= 3)
        def _():
            for h in range(2):
                pltpu.make_async_copy(Ys.at[xslot, h],
                                      out_hbm.at[perm_ref[2 * (k - 3) + h]],
                                      ysem.at[xslot, h]).wait()

        sA = 2 * k
        sB = 2 * k + 1
        pA, slotA = weight_logic(sA)
        for h in range(2):
            pltpu.make_async_copy(x_hbm.at[perm_ref[sA + h]],
                                  Xs.at[xslot, h], xsem.at[xslot, h]).wait()
        compute(sA, xslot, 0, slotA, pA)
        pB, slotB = weight_logic(sB)
        compute(sB, xslot, 1, slotB, pB)
        return carry

    lax.fori_loop(0, NP, step, 0, unroll=False)

    for k in range(NP - 3, NP):
        for h in range(2):
            pltpu.make_async_copy(Ys.at[lax.rem(k, 3), h],
                                  out_hbm.at[perm_ref[2 * k + h]],
                                  ysem.at[lax.rem(k, 3), h]).wait()


def kernel(x, day_indicies, W1, b1, W2, b2, gamma, beta):
    B, T, IN = x.shape
    D, _, HID = W1.shape
    OUT = W2.shape[2]

    day = day_indicies.astype(jnp.int32)
    perm = jnp.argsort(day).astype(jnp.int32)
    sdays = jnp.take(day, perm)

    first = jnp.concatenate(
        [jnp.ones((1,), jnp.int32),
         (sdays[1:] != sdays[:-1]).astype(jnp.int32)])
    ustep = jnp.cumsum(first) - 1
    uday = jnp.zeros((B,), jnp.int32).at[ustep].set(sdays)
    nuniq = jnp.sum(first).reshape(1)

    b1r = b1.reshape(D, 1, HID)
    b2r = b2.reshape(D, 1, OUT)
    gr = gamma.reshape(D, 1, OUT)
    br = beta.reshape(D, 1, OUT)

    vec_spec = pl.BlockSpec(memory_space=pltpu.MemorySpace.VMEM)
    hbm = pl.BlockSpec(memory_space=pltpu.MemorySpace.HBM)

    grid_spec = pltpu.PrefetchScalarGridSpec(
        num_scalar_prefetch=5,
        grid=(1,),
        in_specs=[hbm, hbm, vec_spec, hbm, vec_spec, vec_spec, vec_spec],
        out_specs=hbm,
        scratch_shapes=[
            pltpu.VMEM((3, 2, T, IN), jnp.float32),
            pltpu.VMEM((3, 2, T, OUT), jnp.float32),
            pltpu.VMEM((2, IN, HID), jnp.float32),
            pltpu.VMEM((2, HID, OUT), jnp.float32),
            pltpu.SemaphoreType.DMA((3, 2)),
            pltpu.SemaphoreType.DMA((3, 2)),
            pltpu.SemaphoreType.DMA((2, 2, WCHUNKS)),
        ],
    )

    return pl.pallas_call(
        _body,
        grid_spec=grid_spec,
        out_shape=jax.ShapeDtypeStruct((B, T, OUT), jnp.float32),
        compiler_params=pltpu.CompilerParams(
            dimension_semantics=("arbitrary",),
        ),
    )(perm, ustep, first, uday, nuniq,
      x, W1, b1r, W2, b2r, gr, br)


# speedup vs baseline: 1.1178x; 1.0823x over previous
import jax
import jax.numpy as jnp
from jax import lax
from jax.experimental import pallas as pl
from jax.experimental.pallas import tpu as pltpu

EPS = 1e-5
WCHUNKS = 4


def _w_copy(hbm, vmem, sems, d, slot, midx):
    rows = hbm.shape[1]
    c = rows // WCHUNKS
    return [
        pltpu.make_async_copy(
            hbm.at[d, pl.ds(k * c, c)],
            vmem.at[slot, pl.ds(k * c, c)],
            sems.at[slot, midx, k])
        for k in range(WCHUNKS)
    ]


def _body(perm_ref, ustep_ref, first_ref, uday_ref, nuniq_ref,
          x_hbm, W1_hbm, b1_ref, W2_hbm, b2_ref, g_ref, be_ref, out_hbm,
          Xs, Ys, W1s, W2s, xsem, ysem, wsem):
    B = x_hbm.shape[0]
    nu = nuniq_ref[0]

    pltpu.make_async_copy(x_hbm.at[perm_ref[0]], Xs.at[0], xsem.at[0]).start()
    pltpu.make_async_copy(x_hbm.at[perm_ref[1]], Xs.at[1], xsem.at[1]).start()
    d0 = uday_ref[0]
    for cp in _w_copy(W1_hbm, W1s, wsem, d0, 0, 0):
        cp.start()
    for cp in _w_copy(W2_hbm, W2s, wsem, d0, 0, 1):
        cp.start()

    @pl.when(nu > 1)
    def _():
        d1 = uday_ref[1]
        for cp in _w_copy(W1_hbm, W1s, wsem, d1, 1, 0):
            cp.start()
        for cp in _w_copy(W2_hbm, W2s, wsem, d1, 1, 1):
            cp.start()

    def step(s, carry):
        p = ustep_ref[s]
        slot = lax.rem(p, 3)
        xslot = lax.rem(s, 3)

        @pl.when(s + 2 < B)
        def _():
            pltpu.make_async_copy(x_hbm.at[perm_ref[s + 2]],
                                  Xs.at[lax.rem(s + 2, 3)],
                                  xsem.at[lax.rem(s + 2, 3)]).start()

        is_first = first_ref[s] == 1

        @pl.when(is_first)
        def _():
            d = uday_ref[p]
            for cp in _w_copy(W1_hbm, W1s, wsem, d, slot, 0):
                cp.wait()
            for cp in _w_copy(W2_hbm, W2s, wsem, d, slot, 1):
                cp.wait()

        @pl.when(is_first & (p + 2 < nu))
        def _():
            dn = uday_ref[p + 2]
            nslot = lax.rem(p + 2, 3)
            for cp in _w_copy(W1_hbm, W1s, wsem, dn, nslot, 0):
                cp.start()
            for cp in _w_copy(W2_hbm, W2s, wsem, dn, nslot, 1):
                cp.start()

        pltpu.make_async_copy(x_hbm.at[perm_ref[s]], Xs.at[xslot],
                              xsem.at[xslot]).wait()

        @pl.when(s >= 3)
        def _():
            pltpu.make_async_copy(Ys.at[xslot], out_hbm.at[perm_ref[s - 3]],
                                  ysem.at[xslot]).wait()

        d = uday_ref[p]
        xb = Xs[xslot].astype(jnp.bfloat16)
        h = jnp.dot(xb, W1s[slot].astype(jnp.bfloat16),
                    preferred_element_type=jnp.float32)
        h = jnp.maximum(h + b1_ref[d], 0.0).astype(jnp.bfloat16)
        y = jnp.dot(h, W2s[slot].astype(jnp.bfloat16),
                    preferred_element_type=jnp.float32)
        y = y + b2_ref[d]
        mu = jnp.mean(y, axis=-1, keepdims=True)
        yc = y - mu
        var = jnp.mean(yc * yc, axis=-1, keepdims=True)
        Ys[xslot] = yc * lax.rsqrt(var + EPS) * g_ref[d] + be_ref[d]

        pltpu.make_async_copy(Ys.at[xslot], out_hbm.at[perm_ref[s]],
                              ysem.at[xslot]).start()
        return carry

    lax.fori_loop(0, B, step, 0, unroll=False)

    for k in range(3):
        s = B - 3 + k
        pltpu.make_async_copy(Ys.at[lax.rem(s, 3)],
                              out_hbm.at[perm_ref[s]],
                              ysem.at[lax.rem(s, 3)]).wait()


def kernel(x, day_indicies, W1, b1, W2, b2, gamma, beta):
    B, T, IN = x.shape
    D, _, HID = W1.shape
    OUT = W2.shape[2]

    day = day_indicies.astype(jnp.int32)
    perm = jnp.argsort(day).astype(jnp.int32)
    sdays = jnp.take(day, perm)

    first = jnp.concatenate(
        [jnp.ones((1,), jnp.int32),
         (sdays[1:] != sdays[:-1]).astype(jnp.int32)])
    ustep = jnp.cumsum(first) - 1
    uday = jnp.zeros((B,), jnp.int32).at[ustep].set(sdays)
    nuniq = jnp.sum(first).reshape(1)

    b1r = b1.reshape(D, 1, HID)
    b2r = b2.reshape(D, 1, OUT)
    gr = gamma.reshape(D, 1, OUT)
    br = beta.reshape(D, 1, OUT)

    vec_spec = pl.BlockSpec(memory_space=pltpu.MemorySpace.VMEM)
    hbm = pl.BlockSpec(memory_space=pltpu.MemorySpace.HBM)

    grid_spec = pltpu.PrefetchScalarGridSpec(
        num_scalar_prefetch=5,
        grid=(1,),
        in_specs=[hbm, hbm, vec_spec, hbm, vec_spec, vec_spec, vec_spec],
        out_specs=hbm,
        scratch_shapes=[
            pltpu.VMEM((3, T, IN), jnp.float32),
            pltpu.VMEM((3, T, OUT), jnp.float32),
            pltpu.VMEM((3, IN, HID), jnp.float32),
            pltpu.VMEM((3, HID, OUT), jnp.float32),
            pltpu.SemaphoreType.DMA((3,)),
            pltpu.SemaphoreType.DMA((3,)),
            pltpu.SemaphoreType.DMA((3, 2, WCHUNKS)),
        ],
    )

    return pl.pallas_call(
        _body,
        grid_spec=grid_spec,
        out_shape=jax.ShapeDtypeStruct((B, T, OUT), jnp.float32),
        compiler_params=pltpu.CompilerParams(
            dimension_semantics=("arbitrary",),
        ),
    )(perm, ustep, first, uday, nuniq,
      x, W1, b1r, W2, b2r, gr, br)
